# baseline (speedup 1.0000x reference)
import jax
import jax.numpy as jnp
from jax import lax
from jax.experimental import pallas as pl
from jax.experimental.pallas import tpu as pltpu

N = 32768
M = 1024
CNT = 32
DIM = 256
HEADS = 8
HD = DIM // HEADS
GH = 32
GW = 32
ROW_VOX = GW * CNT
F32 = jnp.float32
BF16 = jnp.bfloat16


def _fused_body(x_ref, qkvw_ref, projw_ref, projb_ref, mask_ref, selt_ref,
                up_ref, out_ref, colsum_ref, q_ref, s_ref):
    t = pl.program_id(0)

    @pl.when((t == 0) | (t == GH))
    def _zero_edge_slot():
        colsum_ref[2] = jnp.zeros((GW, DIM, DIM), BF16)
        s_ref[2] = jnp.zeros((GW, DIM), BF16)

    @pl.when(t < GH)
    def _compute():
        xb = x_ref[...]
        qkv = jnp.dot(xb, qkvw_ref[...], preferred_element_type=F32)
        q = jnp.maximum(qkv[:, :DIM], 0.0).astype(BF16)
        k = jnp.maximum(qkv[:, DIM:2 * DIM], 0.0).astype(BF16)
        v = qkv[:, 2 * DIM:].astype(BF16)
        q_ref[t % 2] = q

        s_ref[t % 3] = jnp.dot(selt_ref[...], k,
                               preferred_element_type=F32).astype(BF16)

        mask = mask_ref[...]
        for m in range(GW):
            lo = max(m - 1, 0) * CNT
            hi = min(m + 2, GW) * CNT
            kvf = lax.dot_general(k[lo:hi], v[lo:hi],
                                  (((0,), (0,)), ((), ())),
                                  preferred_element_type=F32)
            colsum_ref[t % 3, m] = kvf.astype(BF16) * mask

    @pl.when(t >= 1)
    def _emit():
        r = t - 1
        prev_slot = (r + 2) % 3
        cur_slot = r % 3
        next_slot = (r + 1) % 3

        q = q_ref[r % 2]
        s_p = s_ref[prev_slot] + s_ref[cur_slot] + s_ref[next_slot]
        srows = jnp.dot(up_ref[...], s_p, preferred_element_type=F32)
        zden = jnp.dot(q * srows.astype(BF16), mask_ref[...],
                       preferred_element_type=F32) + 1e-6

        ys = []
        for m in range(GW):
            kvp = (colsum_ref[prev_slot, m] + colsum_ref[cur_slot, m]
                   + colsum_ref[next_slot, m])
            qm = q[m * CNT:(m + 1) * CNT]
            ys.append(jnp.dot(qm, kvp, preferred_element_type=F32))
        y = jnp.concatenate(ys, axis=0) / zden
        out_ref[...] = (jnp.dot(y.astype(BF16), projw_ref[...],
                                preferred_element_type=F32)
                        + projb_ref[...])


def kernel(x, qkv_w, proj_w, proj_b, offsets, counts, batch_win_inds,
           batch_win_coords):
    del offsets, counts, batch_win_inds, batch_win_coords

    rg = lax.broadcasted_iota(jnp.int32, (DIM, DIM), 0) // HD
    cg = lax.broadcasted_iota(jnp.int32, (DIM, DIM), 1) // HD
    mask = (rg == cg).astype(BF16)
    mw = lax.broadcasted_iota(jnp.int32, (GW, ROW_VOX), 0)
    rw = lax.broadcasted_iota(jnp.int32, (GW, ROW_VOX), 1) // CNT
    selt = (jnp.abs(mw - rw) <= 1).astype(BF16)
    ri = lax.broadcasted_iota(jnp.int32, (ROW_VOX, GW), 0) // CNT
    ci = lax.broadcasted_iota(jnp.int32, (ROW_VOX, GW), 1)
    up = (ri == ci).astype(BF16)

    out = pl.pallas_call(
        _fused_body,
        grid=(GH + 1,),
        in_specs=[
            pl.BlockSpec((ROW_VOX, DIM),
                         lambda t: (jnp.minimum(t, GH - 1), 0)),
            pl.BlockSpec((DIM, 3 * DIM), lambda t: (0, 0)),
            pl.BlockSpec((DIM, DIM), lambda t: (0, 0)),
            pl.BlockSpec((1, DIM), lambda t: (0, 0)),
            pl.BlockSpec((DIM, DIM), lambda t: (0, 0)),
            pl.BlockSpec((GW, ROW_VOX), lambda t: (0, 0)),
            pl.BlockSpec((ROW_VOX, GW), lambda t: (0, 0)),
        ],
        out_specs=pl.BlockSpec((ROW_VOX, DIM),
                               lambda t: (jnp.maximum(t - 1, 0), 0)),
        out_shape=jax.ShapeDtypeStruct((N, DIM), F32),
        scratch_shapes=[
            pltpu.VMEM((3, GW, DIM, DIM), BF16),
            pltpu.VMEM((2, ROW_VOX, DIM), BF16),
            pltpu.VMEM((3, GW, DIM), BF16),
        ],
    )(x.astype(BF16), qkv_w.astype(BF16), proj_w.astype(BF16),
      proj_b.reshape(1, DIM), mask, selt, up)
    return out


# speedup vs baseline: 1.2044x; 1.2044x over previous
import jax
import jax.numpy as jnp
from jax import lax
from jax.experimental import pallas as pl
from jax.experimental.pallas import tpu as pltpu

N = 32768
M = 1024
CNT = 32
DIM = 256
HEADS = 8
HD = DIM // HEADS
GH = 32
GW = 32
ROW_VOX = GW * CNT
F32 = jnp.float32
BF16 = jnp.bfloat16


def _fused_body(x_ref, qkvw_ref, projw_ref, projb_ref, mask_ref, selt_ref,
                up_ref, out_ref, colsum_ref, q_ref, s_ref):
    t = pl.program_id(0)

    @pl.when((t == 0) | (t == GH))
    def _zero_edge_slot():
        colsum_ref[2] = jnp.zeros((GW, DIM, DIM), BF16)
        s_ref[2] = jnp.zeros((GW, DIM), F32)

    @pl.when(t < GH)
    def _compute():
        xb = x_ref[...]
        qkv = jnp.dot(xb, qkvw_ref[...], preferred_element_type=F32)
        q = jnp.maximum(qkv[:, :DIM], 0.0)
        k = jnp.maximum(qkv[:, DIM:2 * DIM], 0.0)
        v = qkv[:, 2 * DIM:]
        q_ref[t % 2] = q.astype(BF16)

        s_ref[t % 3] = jnp.dot(selt_ref[...], k, preferred_element_type=F32)

        mask = mask_ref[...]
        for m in range(GW):
            lo = max(m - 1, 0) * CNT
            hi = min(m + 2, GW) * CNT
            kvf = lax.dot_general(k[lo:hi], v[lo:hi],
                                  (((0,), (0,)), ((), ())),
                                  preferred_element_type=F32)
            colsum_ref[t % 3, m] = (kvf * mask).astype(BF16)

    @pl.when(t >= 1)
    def _emit():
        r = t - 1
        prev_slot = (r + 2) % 3
        cur_slot = r % 3
        next_slot = (r + 1) % 3

        q = q_ref[r % 2]
        s_p = s_ref[prev_slot] + s_ref[cur_slot] + s_ref[next_slot]
        srows = jnp.dot(up_ref[...], s_p, preferred_element_type=F32)
        zden = jnp.dot(q.astype(F32) * srows, mask_ref[...],
                       preferred_element_type=F32) + 1e-6

        ys = []
        for m in range(GW):
            kvp = (colsum_ref[prev_slot, m] + colsum_ref[cur_slot, m]
                   + colsum_ref[next_slot, m])
            qm = q[m * CNT:(m + 1) * CNT]
            ys.append(jnp.dot(qm, kvp, preferred_element_type=F32))
        y = jnp.concatenate(ys, axis=0) / zden
        out_ref[...] = (jnp.dot(y, projw_ref[...], preferred_element_type=F32)
                        + projb_ref[...])


def kernel(x, qkv_w, proj_w, proj_b, offsets, counts, batch_win_inds,
           batch_win_coords):
    del offsets, counts, batch_win_inds, batch_win_coords

    rg = lax.broadcasted_iota(jnp.int32, (DIM, DIM), 0) // HD
    cg = lax.broadcasted_iota(jnp.int32, (DIM, DIM), 1) // HD
    mask = (rg == cg).astype(F32)
    mw = lax.broadcasted_iota(jnp.int32, (GW, ROW_VOX), 0)
    rw = lax.broadcasted_iota(jnp.int32, (GW, ROW_VOX), 1) // CNT
    selt = (jnp.abs(mw - rw) <= 1).astype(F32)
    ri = lax.broadcasted_iota(jnp.int32, (ROW_VOX, GW), 0) // CNT
    ci = lax.broadcasted_iota(jnp.int32, (ROW_VOX, GW), 1)
    up = (ri == ci).astype(F32)

    out = pl.pallas_call(
        _fused_body,
        grid=(GH + 1,),
        in_specs=[
            pl.BlockSpec((ROW_VOX, DIM),
                         lambda t: (jnp.minimum(t, GH - 1), 0)),
            pl.BlockSpec((DIM, 3 * DIM), lambda t: (0, 0)),
            pl.BlockSpec((DIM, DIM), lambda t: (0, 0)),
            pl.BlockSpec((1, DIM), lambda t: (0, 0)),
            pl.BlockSpec((DIM, DIM), lambda t: (0, 0)),
            pl.BlockSpec((GW, ROW_VOX), lambda t: (0, 0)),
            pl.BlockSpec((ROW_VOX, GW), lambda t: (0, 0)),
        ],
        out_specs=pl.BlockSpec((ROW_VOX, DIM),
                               lambda t: (jnp.maximum(t - 1, 0), 0)),
        out_shape=jax.ShapeDtypeStruct((N, DIM), F32),
        scratch_shapes=[
            pltpu.VMEM((3, GW, DIM, DIM), BF16),
            pltpu.VMEM((2, ROW_VOX, DIM), BF16),
            pltpu.VMEM((3, GW, DIM), F32),
        ],
    )(x, qkv_w, proj_w, proj_b.reshape(1, DIM), mask, selt, up)
    return out
